# baseline (speedup 1.0000x reference)
import functools

import jax
import jax.numpy as jnp
from jax import lax
from jax.experimental import pallas as pl
from jax.experimental.pallas import tpu as pltpu
from jax.experimental.pallas import tpu_sc as plsc

N_ATOMS = 100000
D_IN = 128
D_HID = 128
D_DENSE = 256
MAX_DEG = 10
PER_DEG = 10000
BATCH = 1024

NC = 2
NS = 16
NW = NC * NS
CH = 320
CHP = 64

_mesh = plsc.VectorSubcoreMesh(core_axis_name="c", subcore_axis_name="s")


def _col_off(d, j):
    return ((d - 1) * d // 2 + j) * PER_DEG


def _tile_base(wid):
    base = (wid * (PER_DEG - CH)) // (NW - 1)
    return (base // 8) * 8


@functools.partial(
    pl.kernel,
    out_type=jax.ShapeDtypeStruct((N_ATOMS, D_HID), jnp.float32),
    mesh=_mesh,
    scratch_types=[
        [pltpu.VMEM((CH,), jnp.int32) for _ in range(MAX_DEG)],
        pltpu.VMEM((CH, D_HID), jnp.float32),
        pltpu.SemaphoreType.DMA,
    ],
)
def _nbr_sum(x, cols, out, idx_v, acc_v, sem):
    wid = lax.axis_index("s") * NC + lax.axis_index("c")
    base = _tile_base(wid)
    for d in range(1, MAX_DEG + 1):
        for j in range(d):
            pltpu.sync_copy(cols.at[pl.ds(_col_off(d, j) + base, CH)], idx_v[j])
        for j in range(d):
            pltpu.async_copy(x.at[idx_v[j]], acc_v, sem, add=(j > 0)).wait()
        pltpu.sync_copy(acc_v, out.at[pl.ds((d - 1) * PER_DEG + base, CH)])


@functools.partial(
    pl.kernel,
    out_type=jax.ShapeDtypeStruct((N_ATOMS, D_HID), jnp.float32),
    mesh=_mesh,
    scratch_types=[
        [pltpu.VMEM((CHP,), jnp.int32) for _ in range(MAX_DEG)],
        pltpu.VMEM((CHP, D_HID), jnp.float32),
        [pltpu.VMEM((CHP, D_HID), jnp.float32) for _ in range(MAX_DEG)],
        pltpu.SemaphoreType.DMA,
    ],
)
def _pool_max(x, cols, out, idx_v, acc_v, bufs_v, sem):
    wid = lax.axis_index("s") * NC + lax.axis_index("c")
    base = _tile_base(wid)
    for d in range(1, MAX_DEG + 1):

        def chunk_body(c, _, d=d):
            b2 = base + CHP * c
            dst = (d - 1) * PER_DEG + b2
            for j in range(d):
                pltpu.sync_copy(cols.at[pl.ds(_col_off(d, j) + b2, CHP)], idx_v[j])
            cps = [pltpu.async_copy(x.at[pl.ds(dst, CHP)], acc_v, sem)]
            for j in range(d):
                cps.append(pltpu.async_copy(x.at[idx_v[j]], bufs_v[j], sem))
            for cp in cps:
                cp.wait()

            def row_body(r, _):
                for s in range(D_HID // 16):
                    vals = [acc_v[r, pl.ds(s * 16, 16)]]
                    vals += [bufs_v[j][r, pl.ds(s * 16, 16)] for j in range(d)]
                    while len(vals) > 1:
                        nxt = [jnp.maximum(vals[i], vals[i + 1])
                               for i in range(0, len(vals) - 1, 2)]
                        if len(vals) % 2:
                            nxt.append(vals[-1])
                        vals = nxt
                    acc_v[r, pl.ds(s * 16, 16)] = vals[0]
                return 0

            lax.fori_loop(0, CHP, row_body, 0)
            pltpu.sync_copy(acc_v, out.at[pl.ds(dst, CHP)])
            return 0

        lax.fori_loop(0, CH // CHP, chunk_body, 0)


_SEG_CHUNK = 200
_ROWS_PER_RH = N_ATOMS // 4


@functools.partial(
    pl.kernel,
    out_type=(
        jax.ShapeDtypeStruct((NC, 8, BATCH, 32), jnp.float32),
        jax.ShapeDtypeStruct((4, 8, 2, BATCH, 16), jnp.float32),
    ),
    mesh=_mesh,
    compiler_params=pltpu.CompilerParams(use_tc_tiling_on_sc=False),
    scratch_types=[
        pltpu.VMEM_SHARED((8, BATCH, 32), jnp.float32),
        [pltpu.VMEM((BATCH, 16), jnp.float32) for _ in range(2)],
        pltpu.VMEM((BATCH, 32), jnp.float32),
        pltpu.VMEM((_SEG_CHUNK, 32), jnp.float32),
        pltpu.VMEM((_SEG_CHUNK,), jnp.int32),
    ],
)
def _seg_gather(z, memb, ps_out, pm_out, shared, accm, zerob, zbuf, mvec):
    cid = lax.axis_index("c")
    sid = lax.axis_index("s")
    fs = sid % 8
    rh = cid * 2 + sid // 8

    zeros = jnp.zeros((16,), jnp.float32)
    ninf = jnp.full((16,), -jnp.inf, jnp.float32)

    def init_body(r, _):
        for t in range(2):
            accm[t][r, :] = ninf
            zerob[r, pl.ds(t * 16, 16)] = zeros
        return 0

    lax.fori_loop(0, BATCH, init_body, 0)

    @pl.when(sid == 0)
    def _():
        for f in range(8):
            pltpu.sync_copy(zerob, shared.at[f])

    plsc.subcore_barrier()

    n_chunks = _ROWS_PER_RH // _SEG_CHUNK
    n_grp = (_SEG_CHUNK + 15) // 16

    def chunk_body(k, _):
        r0 = rh * _ROWS_PER_RH + k * _SEG_CHUNK
        pltpu.sync_copy(memb.at[pl.ds(r0, _SEG_CHUNK)], mvec)
        pltpu.sync_copy(z.at[fs, pl.ds(r0, _SEG_CHUNK), :], zbuf)
        pltpu.sync_copy(zbuf, shared.at[fs].at[mvec], add=True)

        def grp_body(g, _):
            off = jnp.minimum(g * 16, _SEG_CHUNK - 16)
            segs = mvec[pl.ds(off, 16)]
            for l in range(16):
                seg = segs[l]
                r = off + l
                for t in range(2):
                    v = zbuf[r, pl.ds(t * 16, 16)]
                    cur = accm[t][seg, :]
                    accm[t][seg, :] = jnp.maximum(cur, v)
            return 0

        lax.fori_loop(0, n_grp, grp_body, 0)
        return 0

    lax.fori_loop(0, n_chunks, chunk_body, 0)

    for t in range(2):
        pltpu.sync_copy(accm[t], pm_out.at[rh, fs, t])
    plsc.subcore_barrier()

    @pl.when(sid == 0)
    def _():
        for f in range(8):
            pltpu.sync_copy(shared.at[f], ps_out.at[cid, f])


_BR = 2000


def _gc_body(xb, rb, wr, ws, bb, sc, sh, ob):
    acc = jnp.dot(rb[...], wr[0], preferred_element_type=jnp.float32)
    acc = acc + jnp.dot(xb[...], ws[0], preferred_element_type=jnp.float32)
    t = jnp.tanh(acc + bb[0, 0])
    ob[...] = t * sc[0] + sh[0]


def _gc_matmul(x, rel, wr, ws, bsum, scale, shift):
    nb = PER_DEG // _BR
    return pl.pallas_call(
        _gc_body,
        grid=(MAX_DEG, nb),
        in_specs=[
            pl.BlockSpec((_BR, D_HID), lambda d, b: (d * nb + b, 0)),
            pl.BlockSpec((_BR, D_HID), lambda d, b: (d * nb + b, 0)),
            pl.BlockSpec((1, D_HID, D_HID), lambda d, b: (d, 0, 0)),
            pl.BlockSpec((1, D_HID, D_HID), lambda d, b: (d, 0, 0)),
            pl.BlockSpec((1, 1, D_HID), lambda d, b: (d, 0, 0)),
            pl.BlockSpec((1, D_HID), lambda d, b: (0, 0)),
            pl.BlockSpec((1, D_HID), lambda d, b: (0, 0)),
        ],
        out_specs=pl.BlockSpec((_BR, D_HID), lambda d, b: (d * nb + b, 0)),
        out_shape=jax.ShapeDtypeStruct((N_ATOMS, D_HID), jnp.float32),
    )(x, rel, wr, ws, bsum, scale, shift)


def _dense_body(xb, w, bb, sc, sh, ob):
    acc = jnp.dot(xb[...], w[...], preferred_element_type=jnp.float32)
    t = jnp.tanh(acc + bb[0]) * sc[0] + sh[0]
    for f in range(8):
        ob[f] = t[:, f * 32:(f + 1) * 32]


def _dense(x, w, b, scale, shift):
    nb = N_ATOMS // _BR
    return pl.pallas_call(
        _dense_body,
        grid=(nb,),
        in_specs=[
            pl.BlockSpec((_BR, D_HID), lambda b: (b, 0)),
            pl.BlockSpec((D_HID, D_DENSE), lambda b: (0, 0)),
            pl.BlockSpec((1, D_DENSE), lambda b: (0, 0)),
            pl.BlockSpec((1, D_DENSE), lambda b: (0, 0)),
            pl.BlockSpec((1, D_DENSE), lambda b: (0, 0)),
        ],
        out_specs=pl.BlockSpec((8, _BR, 32), lambda b: (0, b, 0)),
        out_shape=jax.ShapeDtypeStruct((8, N_ATOMS, 32), jnp.float32),
    )(x, w, b, scale, shift)


_ASM = 8 * BATCH * 32


def _assemble_body(ps, pm, os_, om_):
    s = ps[0] + ps[1]
    m = jnp.maximum(jnp.maximum(pm[0], pm[1]), jnp.maximum(pm[2], pm[3]))
    os_[...] = jnp.tanh(s)[None]
    om_[...] = jnp.tanh(m)[None]


def _assemble(ps, pm):
    blk = _ASM // 8
    return pl.pallas_call(
        _assemble_body,
        grid=(8,),
        in_specs=[
            pl.BlockSpec((NC, blk), lambda b: (0, b)),
            pl.BlockSpec((4, blk), lambda b: (0, b)),
        ],
        out_specs=[
            pl.BlockSpec((1, blk), lambda b: (0, b)),
            pl.BlockSpec((1, blk), lambda b: (0, b)),
        ],
        out_shape=[
            jax.ShapeDtypeStruct((1, _ASM), jnp.float32),
            jax.ShapeDtypeStruct((1, _ASM), jnp.float32),
        ],
    )(ps.reshape(NC, _ASM), pm.reshape(4, _ASM))


def _bn_fold(gamma, beta, mean, var, eps=1e-3):
    scale = gamma / jnp.sqrt(var + eps)
    shift = beta - mean * scale
    return scale.reshape(1, -1), shift.reshape(1, -1)


def kernel(atom_features, degree_slice, membership,
           deg_adj_1, deg_adj_2, deg_adj_3, deg_adj_4, deg_adj_5,
           deg_adj_6, deg_adj_7, deg_adj_8, deg_adj_9, deg_adj_10,
           gc1_W, gc1_b, gc2_W, gc2_b,
           bn1_gamma, bn1_beta, bn1_mean, bn1_var,
           dense_W, dense_b,
           bn3_gamma, bn3_beta, bn3_mean, bn3_var):
    del degree_slice
    adjs = [deg_adj_1, deg_adj_2, deg_adj_3, deg_adj_4, deg_adj_5,
            deg_adj_6, deg_adj_7, deg_adj_8, deg_adj_9, deg_adj_10]
    cols = jnp.concatenate([a.T.reshape(-1) for a in adjs])

    w1r, w1s = gc1_W[0:2 * MAX_DEG:2], gc1_W[1:2 * MAX_DEG:2]
    b1 = (gc1_b[0:2 * MAX_DEG:2] + gc1_b[1:2 * MAX_DEG:2]).reshape(MAX_DEG, 1, D_HID)
    w2r, w2s = gc2_W[0:2 * MAX_DEG:2], gc2_W[1:2 * MAX_DEG:2]
    b2 = (gc2_b[0:2 * MAX_DEG:2] + gc2_b[1:2 * MAX_DEG:2]).reshape(MAX_DEG, 1, D_HID)
    sc1, sh1 = _bn_fold(bn1_gamma, bn1_beta, bn1_mean, bn1_var)
    sc3, sh3 = _bn_fold(bn3_gamma, bn3_beta, bn3_mean, bn3_var)

    rel1 = _nbr_sum(atom_features, cols)
    y1 = _gc_matmul(atom_features, rel1, w1r, w1s, b1, sc1, sh1)
    p1 = _pool_max(y1, cols)
    rel2 = _nbr_sum(p1, cols)
    y2 = _gc_matmul(p1, rel2, w2r, w2s, b2, sc1, sh1)
    p2 = _pool_max(y2, cols)
    z = _dense(p2, dense_W, dense_b.reshape(1, -1), sc3, sh3)
    ps, pm = _seg_gather(z, membership)
    out_s, out_m = _assemble(ps, pm)
    out_s = out_s.reshape(8, BATCH, 32).transpose(1, 0, 2).reshape(BATCH, D_DENSE)
    out_m = out_m.reshape(8, 2, BATCH, 16).transpose(2, 0, 1, 3).reshape(BATCH, D_DENSE)
    return jnp.concatenate([out_s, out_m], axis=1)


# speedup vs baseline: 1.0868x; 1.0868x over previous
import functools

import jax
import jax.numpy as jnp
from jax import lax
from jax.experimental import pallas as pl
from jax.experimental.pallas import tpu as pltpu
from jax.experimental.pallas import tpu_sc as plsc

N_ATOMS = 100000
D_IN = 128
D_HID = 128
D_DENSE = 256
MAX_DEG = 10
PER_DEG = 10000
BATCH = 1024

NC = 2
NS = 16
NW = NC * NS
CH = 320
CHP = 32

_mesh = plsc.VectorSubcoreMesh(core_axis_name="c", subcore_axis_name="s")


def _col_off(d, j):
    return ((d - 1) * d // 2 + j) * PER_DEG


def _tile_base(wid):
    base = (wid * (PER_DEG - CH)) // (NW - 1)
    return (base // 8) * 8


@functools.partial(
    pl.kernel,
    out_type=jax.ShapeDtypeStruct((N_ATOMS, D_HID), jnp.float32),
    mesh=_mesh,
    scratch_types=[
        [pltpu.VMEM((CH,), jnp.int32) for _ in range(MAX_DEG)],
        pltpu.VMEM((CH, D_HID), jnp.float32),
        pltpu.SemaphoreType.DMA,
    ],
)
def _nbr_sum(x, cols, out, idx_v, acc_v, sem):
    wid = lax.axis_index("s") * NC + lax.axis_index("c")
    base = _tile_base(wid)
    for d in range(1, MAX_DEG + 1):
        for j in range(d):
            pltpu.sync_copy(cols.at[pl.ds(_col_off(d, j) + base, CH)], idx_v[j])
        for j in range(d):
            pltpu.async_copy(x.at[idx_v[j]], acc_v, sem, add=(j > 0)).wait()
        pltpu.sync_copy(acc_v, out.at[pl.ds((d - 1) * PER_DEG + base, CH)])


@functools.partial(
    pl.kernel,
    out_type=jax.ShapeDtypeStruct((N_ATOMS, D_HID), jnp.float32),
    mesh=_mesh,
    scratch_types=[
        [[pltpu.VMEM((CHP,), jnp.int32) for _ in range(MAX_DEG)] for _ in range(2)],
        [pltpu.VMEM((CHP, D_HID), jnp.float32) for _ in range(2)],
        [[pltpu.VMEM((CHP, D_HID), jnp.float32) for _ in range(MAX_DEG)]
         for _ in range(2)],
        [pltpu.SemaphoreType.DMA for _ in range(2)],
    ],
)
def _pool_max(x, cols, out, idx_v, acc_v, bufs_v, sem):
    wid = lax.axis_index("s") * NC + lax.axis_index("c")
    base = _tile_base(wid)
    npair = CH // CHP // 2

    for d in range(1, MAX_DEG + 1):

        def issue(c, p, d=d):
            b2 = base + CHP * c
            dst = (d - 1) * PER_DEG + b2
            for j in range(d):
                pltpu.sync_copy(cols.at[pl.ds(_col_off(d, j) + b2, CHP)],
                                idx_v[p][j])
            pltpu.async_copy(x.at[pl.ds(dst, CHP)], acc_v[p], sem[p])
            for j in range(d):
                pltpu.async_copy(x.at[idx_v[p][j]], bufs_v[p][j], sem[p])

        def consume(c, p, d=d):
            pltpu.make_async_copy(x.at[pl.ds(0, CHP)], acc_v[p], sem[p]).wait()
            for j in range(d):
                pltpu.make_async_copy(x.at[idx_v[p][j]], bufs_v[p][j],
                                      sem[p]).wait()

            def row_body(r, _):
                for s in range(D_HID // 16):
                    vals = [acc_v[p][r, pl.ds(s * 16, 16)]]
                    vals += [bufs_v[p][j][r, pl.ds(s * 16, 16)] for j in range(d)]
                    while len(vals) > 1:
                        nxt = [jnp.maximum(vals[i], vals[i + 1])
                               for i in range(0, len(vals) - 1, 2)]
                        if len(vals) % 2:
                            nxt.append(vals[-1])
                        vals = nxt
                    acc_v[p][r, pl.ds(s * 16, 16)] = vals[0]
                return 0

            lax.fori_loop(0, CHP, row_body, 0)
            dst = (d - 1) * PER_DEG + base + CHP * c
            pltpu.sync_copy(acc_v[p], out.at[pl.ds(dst, CHP)])

        issue(0, 0)

        def pair_body(g, _, d=d):
            issue(2 * g + 1, 1)
            consume(2 * g, 0)

            @pl.when(g < npair - 1)
            def _():
                issue(2 * g + 2, 0)

            consume(2 * g + 1, 1)
            return 0

        lax.fori_loop(0, npair, pair_body, 0)


_SEG_CHUNK = 200
_ROWS_PER_RH = N_ATOMS // 4


@functools.partial(
    pl.kernel,
    out_type=(
        jax.ShapeDtypeStruct((NC, 8, BATCH, 32), jnp.float32),
        jax.ShapeDtypeStruct((4, 8, 2, BATCH, 16), jnp.float32),
    ),
    mesh=_mesh,
    compiler_params=pltpu.CompilerParams(use_tc_tiling_on_sc=False),
    scratch_types=[
        pltpu.VMEM_SHARED((8, BATCH, 32), jnp.float32),
        [pltpu.VMEM((BATCH, 16), jnp.float32) for _ in range(2)],
        pltpu.VMEM((BATCH, 32), jnp.float32),
        [pltpu.VMEM((_SEG_CHUNK, 32), jnp.float32) for _ in range(2)],
        [pltpu.VMEM((_SEG_CHUNK,), jnp.int32) for _ in range(2)],
        [pltpu.SemaphoreType.DMA for _ in range(2)],
    ],
)
def _seg_gather(z, memb, ps_out, pm_out, shared, accm, zerob, zbuf, mvec, sem):
    cid = lax.axis_index("c")
    sid = lax.axis_index("s")
    fs = sid % 8
    rh = cid * 2 + sid // 8

    zeros = jnp.zeros((16,), jnp.float32)
    ninf = jnp.full((16,), -jnp.inf, jnp.float32)

    def init_body(r, _):
        for t in range(2):
            accm[t][r, :] = ninf
            zerob[r, pl.ds(t * 16, 16)] = zeros
        return 0

    lax.fori_loop(0, BATCH, init_body, 0)

    @pl.when(sid == 0)
    def _():
        for f in range(8):
            pltpu.sync_copy(zerob, shared.at[f])

    plsc.subcore_barrier()

    n_chunks = _ROWS_PER_RH // _SEG_CHUNK
    n_grp = (_SEG_CHUNK + 15) // 16

    def issue(k, p):
        r0 = rh * _ROWS_PER_RH + k * _SEG_CHUNK
        pltpu.async_copy(memb.at[pl.ds(r0, _SEG_CHUNK)], mvec[p], sem[p])
        pltpu.async_copy(z.at[fs, pl.ds(r0, _SEG_CHUNK), :], zbuf[p], sem[p])

    def consume(k, p):
        pltpu.make_async_copy(memb.at[pl.ds(0, _SEG_CHUNK)], mvec[p],
                              sem[p]).wait()
        pltpu.make_async_copy(z.at[fs, pl.ds(0, _SEG_CHUNK), :], zbuf[p],
                              sem[p]).wait()
        pltpu.sync_copy(zbuf[p], shared.at[fs].at[mvec[p]], add=True)

        def grp_body(g, _):
            off = jnp.minimum(g * 16, _SEG_CHUNK - 16)
            segs = mvec[p][pl.ds(off, 16)]
            for l in range(16):
                seg = segs[l]
                r = off + l
                for t in range(2):
                    v = zbuf[p][r, pl.ds(t * 16, 16)]
                    cur = accm[t][seg, :]
                    accm[t][seg, :] = jnp.maximum(cur, v)
            return 0

        lax.fori_loop(0, n_grp, grp_body, 0)

    issue(0, 0)

    def pair_body(g, _):
        issue(2 * g + 1, 1)
        consume(2 * g, 0)
        issue(2 * g + 2, 0)
        consume(2 * g + 1, 1)
        return 0

    lax.fori_loop(0, (n_chunks - 1) // 2, pair_body, 0)
    consume(n_chunks - 1, 0)

    for t in range(2):
        pltpu.sync_copy(accm[t], pm_out.at[rh, fs, t])
    plsc.subcore_barrier()

    @pl.when(sid == 0)
    def _():
        for f in range(8):
            pltpu.sync_copy(shared.at[f], ps_out.at[cid, f])


_BR = 2000


def _gc_body(xb, rb, wr, ws, bb, sc, sh, ob):
    acc = jnp.dot(rb[...], wr[0], preferred_element_type=jnp.float32)
    acc = acc + jnp.dot(xb[...], ws[0], preferred_element_type=jnp.float32)
    t = jnp.tanh(acc + bb[0, 0])
    ob[...] = t * sc[0] + sh[0]


def _gc_matmul(x, rel, wr, ws, bsum, scale, shift):
    nb = PER_DEG // _BR
    return pl.pallas_call(
        _gc_body,
        grid=(MAX_DEG, nb),
        in_specs=[
            pl.BlockSpec((_BR, D_HID), lambda d, b: (d * nb + b, 0)),
            pl.BlockSpec((_BR, D_HID), lambda d, b: (d * nb + b, 0)),
            pl.BlockSpec((1, D_HID, D_HID), lambda d, b: (d, 0, 0)),
            pl.BlockSpec((1, D_HID, D_HID), lambda d, b: (d, 0, 0)),
            pl.BlockSpec((1, 1, D_HID), lambda d, b: (d, 0, 0)),
            pl.BlockSpec((1, D_HID), lambda d, b: (0, 0)),
            pl.BlockSpec((1, D_HID), lambda d, b: (0, 0)),
        ],
        out_specs=pl.BlockSpec((_BR, D_HID), lambda d, b: (d * nb + b, 0)),
        out_shape=jax.ShapeDtypeStruct((N_ATOMS, D_HID), jnp.float32),
    )(x, rel, wr, ws, bsum, scale, shift)


def _dense_body(xb, w, bb, sc, sh, ob):
    acc = jnp.dot(xb[...], w[...], preferred_element_type=jnp.float32)
    t = jnp.tanh(acc + bb[0]) * sc[0] + sh[0]
    for f in range(8):
        ob[f] = t[:, f * 32:(f + 1) * 32]


def _dense(x, w, b, scale, shift):
    nb = N_ATOMS // _BR
    return pl.pallas_call(
        _dense_body,
        grid=(nb,),
        in_specs=[
            pl.BlockSpec((_BR, D_HID), lambda b: (b, 0)),
            pl.BlockSpec((D_HID, D_DENSE), lambda b: (0, 0)),
            pl.BlockSpec((1, D_DENSE), lambda b: (0, 0)),
            pl.BlockSpec((1, D_DENSE), lambda b: (0, 0)),
            pl.BlockSpec((1, D_DENSE), lambda b: (0, 0)),
        ],
        out_specs=pl.BlockSpec((8, _BR, 32), lambda b: (0, b, 0)),
        out_shape=jax.ShapeDtypeStruct((8, N_ATOMS, 32), jnp.float32),
    )(x, w, b, scale, shift)


_ASM = 8 * BATCH * 32


def _assemble_body(ps, pm, os_, om_):
    s = ps[0] + ps[1]
    m = jnp.maximum(jnp.maximum(pm[0], pm[1]), jnp.maximum(pm[2], pm[3]))
    os_[...] = jnp.tanh(s)[None]
    om_[...] = jnp.tanh(m)[None]


def _assemble(ps, pm):
    blk = _ASM // 8
    return pl.pallas_call(
        _assemble_body,
        grid=(8,),
        in_specs=[
            pl.BlockSpec((NC, blk), lambda b: (0, b)),
            pl.BlockSpec((4, blk), lambda b: (0, b)),
        ],
        out_specs=[
            pl.BlockSpec((1, blk), lambda b: (0, b)),
            pl.BlockSpec((1, blk), lambda b: (0, b)),
        ],
        out_shape=[
            jax.ShapeDtypeStruct((1, _ASM), jnp.float32),
            jax.ShapeDtypeStruct((1, _ASM), jnp.float32),
        ],
    )(ps.reshape(NC, _ASM), pm.reshape(4, _ASM))


def _bn_fold(gamma, beta, mean, var, eps=1e-3):
    scale = gamma / jnp.sqrt(var + eps)
    shift = beta - mean * scale
    return scale.reshape(1, -1), shift.reshape(1, -1)


def kernel(atom_features, degree_slice, membership,
           deg_adj_1, deg_adj_2, deg_adj_3, deg_adj_4, deg_adj_5,
           deg_adj_6, deg_adj_7, deg_adj_8, deg_adj_9, deg_adj_10,
           gc1_W, gc1_b, gc2_W, gc2_b,
           bn1_gamma, bn1_beta, bn1_mean, bn1_var,
           dense_W, dense_b,
           bn3_gamma, bn3_beta, bn3_mean, bn3_var):
    del degree_slice
    adjs = [deg_adj_1, deg_adj_2, deg_adj_3, deg_adj_4, deg_adj_5,
            deg_adj_6, deg_adj_7, deg_adj_8, deg_adj_9, deg_adj_10]
    cols = jnp.concatenate([a.T.reshape(-1) for a in adjs])

    w1r, w1s = gc1_W[0:2 * MAX_DEG:2], gc1_W[1:2 * MAX_DEG:2]
    b1 = (gc1_b[0:2 * MAX_DEG:2] + gc1_b[1:2 * MAX_DEG:2]).reshape(MAX_DEG, 1, D_HID)
    w2r, w2s = gc2_W[0:2 * MAX_DEG:2], gc2_W[1:2 * MAX_DEG:2]
    b2 = (gc2_b[0:2 * MAX_DEG:2] + gc2_b[1:2 * MAX_DEG:2]).reshape(MAX_DEG, 1, D_HID)
    sc1, sh1 = _bn_fold(bn1_gamma, bn1_beta, bn1_mean, bn1_var)
    sc3, sh3 = _bn_fold(bn3_gamma, bn3_beta, bn3_mean, bn3_var)

    rel1 = _nbr_sum(atom_features, cols)
    y1 = _gc_matmul(atom_features, rel1, w1r, w1s, b1, sc1, sh1)
    p1 = _pool_max(y1, cols)
    rel2 = _nbr_sum(p1, cols)
    y2 = _gc_matmul(p1, rel2, w2r, w2s, b2, sc1, sh1)
    p2 = _pool_max(y2, cols)
    z = _dense(p2, dense_W, dense_b.reshape(1, -1), sc3, sh3)
    ps, pm = _seg_gather(z, membership)
    out_s, out_m = _assemble(ps, pm)
    out_s = out_s.reshape(8, BATCH, 32).transpose(1, 0, 2).reshape(BATCH, D_DENSE)
    out_m = out_m.reshape(8, 2, BATCH, 16).transpose(2, 0, 1, 3).reshape(BATCH, D_DENSE)
    return jnp.concatenate([out_s, out_m], axis=1)


# speedup vs baseline: 1.1939x; 1.0985x over previous
import functools

import jax
import jax.numpy as jnp
from jax import lax
from jax.experimental import pallas as pl
from jax.experimental.pallas import tpu as pltpu
from jax.experimental.pallas import tpu_sc as plsc

N_ATOMS = 100000
D_IN = 128
D_HID = 128
D_DENSE = 256
MAX_DEG = 10
PER_DEG = 10000
BATCH = 1024

NC = 2
NS = 16
NW = NC * NS
CH = 320
CHP = 32
_CHP_TBL = {1: 80, 2: 80, 3: 80, 4: 80, 5: 64, 6: 40, 7: 40, 8: 40, 9: 32, 10: 32}

_mesh = plsc.VectorSubcoreMesh(core_axis_name="c", subcore_axis_name="s")


def _col_off(d, j):
    return ((d - 1) * d // 2 + j) * PER_DEG


def _tile_base(wid):
    base = (wid * (PER_DEG - CH)) // (NW - 1)
    return (base // 8) * 8


@functools.partial(
    pl.kernel,
    out_type=jax.ShapeDtypeStruct((N_ATOMS, D_HID), jnp.float32),
    mesh=_mesh,
    scratch_types=[
        [pltpu.VMEM((CH,), jnp.int32) for _ in range(MAX_DEG)],
        pltpu.VMEM((CH, D_HID), jnp.float32),
        pltpu.SemaphoreType.DMA,
    ],
)
def _nbr_sum(x, cols, out, idx_v, acc_v, sem):
    wid = lax.axis_index("s") * NC + lax.axis_index("c")
    base = _tile_base(wid)
    for d in range(1, MAX_DEG + 1):
        for j in range(d):
            pltpu.sync_copy(cols.at[pl.ds(_col_off(d, j) + base, CH)], idx_v[j])
        for j in range(d):
            pltpu.async_copy(x.at[idx_v[j]], acc_v, sem, add=(j > 0)).wait()
        pltpu.sync_copy(acc_v, out.at[pl.ds((d - 1) * PER_DEG + base, CH)])


@functools.partial(
    pl.kernel,
    out_type=jax.ShapeDtypeStruct((N_ATOMS, D_HID), jnp.float32),
    mesh=_mesh,
    scratch_types=[
        [[pltpu.VMEM((_CHP_TBL[d] * d,), jnp.int32) for d in range(1, MAX_DEG + 1)]
         for _ in range(2)],
        [pltpu.VMEM((CH, D_HID), jnp.float32) for _ in range(2)],
        [pltpu.VMEM((80, D_HID), jnp.float32) for _ in range(2)],
        [pltpu.SemaphoreType.DMA for _ in range(2)],
    ],
)
def _pool_max(x, colsr, out, idx_v, bufs_v, acc_v, sem):
    wid = lax.axis_index("s") * NC + lax.axis_index("c")
    base = _tile_base(wid)

    for d in range(1, MAX_DEG + 1):
        chp = _CHP_TBL[d]
        n = CH // chp
        sz = chp * d
        roff = _col_off(d, 0)

        def bdst(p, sz=sz):
            return bufs_v[p] if sz == CH else bufs_v[p].at[pl.ds(0, sz)]

        def adst(p, chp=chp):
            return acc_v[p] if chp == 80 else acc_v[p].at[pl.ds(0, chp)]

        def issue(c, p, d=d, chp=chp, sz=sz, roff=roff):
            b2 = base + chp * c
            pltpu.sync_copy(colsr.at[pl.ds(roff + b2 * d, sz)], idx_v[p][d - 1])
            pltpu.async_copy(x.at[pl.ds((d - 1) * PER_DEG + b2, chp)],
                             adst(p), sem[p])
            pltpu.async_copy(x.at[idx_v[p][d - 1]], bdst(p), sem[p])

        def consume(c, p, d=d, chp=chp):
            pltpu.make_async_copy(x.at[pl.ds(0, chp)], adst(p), sem[p]).wait()
            pltpu.make_async_copy(x.at[idx_v[p][d - 1]], bdst(p), sem[p]).wait()

            def row_body(r, _):
                for s in range(D_HID // 16):
                    ds_ = pl.ds(s * 16, 16)
                    vals = [acc_v[p][r, ds_]]
                    vals += [bufs_v[p][r * d + j, ds_] for j in range(d)]
                    while len(vals) > 1:
                        nxt = [jnp.maximum(vals[i], vals[i + 1])
                               for i in range(0, len(vals) - 1, 2)]
                        if len(vals) % 2:
                            nxt.append(vals[-1])
                        vals = nxt
                    acc_v[p][r, ds_] = vals[0]
                return 0

            lax.fori_loop(0, chp, row_body, 0)
            pltpu.sync_copy(adst(p),
                            out.at[pl.ds((d - 1) * PER_DEG + base + chp * c, chp)])

        issue(0, 0)
        if n % 2 == 0:
            def pair_body(g, _, issue=issue, consume=consume, n=n):
                issue(2 * g + 1, 1)
                consume(2 * g, 0)

                @pl.when(g < n // 2 - 1)
                def _():
                    issue(2 * g + 2, 0)

                consume(2 * g + 1, 1)
                return 0

            lax.fori_loop(0, n // 2, pair_body, 0)
        else:
            def pair_body(g, _, issue=issue, consume=consume):
                issue(2 * g + 1, 1)
                consume(2 * g, 0)
                issue(2 * g + 2, 0)
                consume(2 * g + 1, 1)
                return 0

            lax.fori_loop(0, (n - 1) // 2, pair_body, 0)
            consume(n - 1, 0)


_SEG_CHUNK = 200
_ROWS_PER_RH = N_ATOMS // 4


@functools.partial(
    pl.kernel,
    out_type=(
        jax.ShapeDtypeStruct((NC, 8, BATCH, 32), jnp.float32),
        jax.ShapeDtypeStruct((4, 8, 2, BATCH, 16), jnp.float32),
    ),
    mesh=_mesh,
    compiler_params=pltpu.CompilerParams(use_tc_tiling_on_sc=False),
    scratch_types=[
        pltpu.VMEM_SHARED((8, BATCH, 32), jnp.float32),
        [pltpu.VMEM((BATCH, 16), jnp.float32) for _ in range(2)],
        pltpu.VMEM((BATCH, 32), jnp.float32),
        [pltpu.VMEM((_SEG_CHUNK, 32), jnp.float32) for _ in range(2)],
        [pltpu.VMEM((_SEG_CHUNK,), jnp.int32) for _ in range(2)],
        [pltpu.SemaphoreType.DMA for _ in range(2)],
    ],
)
def _seg_gather(z, memb, ps_out, pm_out, shared, accm, zerob, zbuf, mvec, sem):
    cid = lax.axis_index("c")
    sid = lax.axis_index("s")
    fs = sid % 8
    rh = cid * 2 + sid // 8

    zeros = jnp.zeros((16,), jnp.float32)
    ninf = jnp.full((16,), -jnp.inf, jnp.float32)

    def init_body(r, _):
        for t in range(2):
            accm[t][r, :] = ninf
            zerob[r, pl.ds(t * 16, 16)] = zeros
        return 0

    lax.fori_loop(0, BATCH, init_body, 0)

    @pl.when(sid == 0)
    def _():
        for f in range(8):
            pltpu.sync_copy(zerob, shared.at[f])

    plsc.subcore_barrier()

    n_chunks = _ROWS_PER_RH // _SEG_CHUNK
    n_grp = (_SEG_CHUNK + 15) // 16

    def issue(k, p):
        r0 = rh * _ROWS_PER_RH + k * _SEG_CHUNK
        pltpu.async_copy(memb.at[pl.ds(r0, _SEG_CHUNK)], mvec[p], sem[p])
        pltpu.async_copy(z.at[fs, pl.ds(r0, _SEG_CHUNK), :], zbuf[p], sem[p])

    def consume(k, p):
        pltpu.make_async_copy(memb.at[pl.ds(0, _SEG_CHUNK)], mvec[p],
                              sem[p]).wait()
        pltpu.make_async_copy(z.at[fs, pl.ds(0, _SEG_CHUNK), :], zbuf[p],
                              sem[p]).wait()
        pltpu.sync_copy(zbuf[p], shared.at[fs].at[mvec[p]], add=True)

        def grp_body(g, _):
            off = jnp.minimum(g * 16, _SEG_CHUNK - 16)
            segs = mvec[p][pl.ds(off, 16)]
            for l in range(16):
                seg = segs[l]
                r = off + l
                for t in range(2):
                    v = zbuf[p][r, pl.ds(t * 16, 16)]
                    cur = accm[t][seg, :]
                    accm[t][seg, :] = jnp.maximum(cur, v)
            return 0

        lax.fori_loop(0, n_grp, grp_body, 0)

    issue(0, 0)

    def pair_body(g, _):
        issue(2 * g + 1, 1)
        consume(2 * g, 0)
        issue(2 * g + 2, 0)
        consume(2 * g + 1, 1)
        return 0

    lax.fori_loop(0, (n_chunks - 1) // 2, pair_body, 0)
    consume(n_chunks - 1, 0)

    for t in range(2):
        pltpu.sync_copy(accm[t], pm_out.at[rh, fs, t])
    plsc.subcore_barrier()

    @pl.when(sid == 0)
    def _():
        for f in range(8):
            pltpu.sync_copy(shared.at[f], ps_out.at[cid, f])


_BR = 2000


def _gc_body(xb, rb, wr, ws, bb, sc, sh, ob):
    acc = jnp.dot(rb[...], wr[0], preferred_element_type=jnp.float32)
    acc = acc + jnp.dot(xb[...], ws[0], preferred_element_type=jnp.float32)
    t = jnp.tanh(acc + bb[0, 0])
    ob[...] = t * sc[0] + sh[0]


def _gc_matmul(x, rel, wr, ws, bsum, scale, shift):
    nb = PER_DEG // _BR
    return pl.pallas_call(
        _gc_body,
        grid=(MAX_DEG, nb),
        in_specs=[
            pl.BlockSpec((_BR, D_HID), lambda d, b: (d * nb + b, 0)),
            pl.BlockSpec((_BR, D_HID), lambda d, b: (d * nb + b, 0)),
            pl.BlockSpec((1, D_HID, D_HID), lambda d, b: (d, 0, 0)),
            pl.BlockSpec((1, D_HID, D_HID), lambda d, b: (d, 0, 0)),
            pl.BlockSpec((1, 1, D_HID), lambda d, b: (d, 0, 0)),
            pl.BlockSpec((1, D_HID), lambda d, b: (0, 0)),
            pl.BlockSpec((1, D_HID), lambda d, b: (0, 0)),
        ],
        out_specs=pl.BlockSpec((_BR, D_HID), lambda d, b: (d * nb + b, 0)),
        out_shape=jax.ShapeDtypeStruct((N_ATOMS, D_HID), jnp.float32),
    )(x, rel, wr, ws, bsum, scale, shift)


def _dense_body(xb, w, bb, sc, sh, ob):
    acc = jnp.dot(xb[...], w[...], preferred_element_type=jnp.float32)
    t = jnp.tanh(acc + bb[0]) * sc[0] + sh[0]
    for f in range(8):
        ob[f] = t[:, f * 32:(f + 1) * 32]


def _dense(x, w, b, scale, shift):
    nb = N_ATOMS // _BR
    return pl.pallas_call(
        _dense_body,
        grid=(nb,),
        in_specs=[
            pl.BlockSpec((_BR, D_HID), lambda b: (b, 0)),
            pl.BlockSpec((D_HID, D_DENSE), lambda b: (0, 0)),
            pl.BlockSpec((1, D_DENSE), lambda b: (0, 0)),
            pl.BlockSpec((1, D_DENSE), lambda b: (0, 0)),
            pl.BlockSpec((1, D_DENSE), lambda b: (0, 0)),
        ],
        out_specs=pl.BlockSpec((8, _BR, 32), lambda b: (0, b, 0)),
        out_shape=jax.ShapeDtypeStruct((8, N_ATOMS, 32), jnp.float32),
    )(x, w, b, scale, shift)


_ASM = 8 * BATCH * 32


def _assemble_body(ps, pm, os_, om_):
    s = ps[0] + ps[1]
    m = jnp.maximum(jnp.maximum(pm[0], pm[1]), jnp.maximum(pm[2], pm[3]))
    os_[...] = jnp.tanh(s)[None]
    om_[...] = jnp.tanh(m)[None]


def _assemble(ps, pm):
    blk = _ASM // 8
    return pl.pallas_call(
        _assemble_body,
        grid=(8,),
        in_specs=[
            pl.BlockSpec((NC, blk), lambda b: (0, b)),
            pl.BlockSpec((4, blk), lambda b: (0, b)),
        ],
        out_specs=[
            pl.BlockSpec((1, blk), lambda b: (0, b)),
            pl.BlockSpec((1, blk), lambda b: (0, b)),
        ],
        out_shape=[
            jax.ShapeDtypeStruct((1, _ASM), jnp.float32),
            jax.ShapeDtypeStruct((1, _ASM), jnp.float32),
        ],
    )(ps.reshape(NC, _ASM), pm.reshape(4, _ASM))


def _bn_fold(gamma, beta, mean, var, eps=1e-3):
    scale = gamma / jnp.sqrt(var + eps)
    shift = beta - mean * scale
    return scale.reshape(1, -1), shift.reshape(1, -1)


def kernel(atom_features, degree_slice, membership,
           deg_adj_1, deg_adj_2, deg_adj_3, deg_adj_4, deg_adj_5,
           deg_adj_6, deg_adj_7, deg_adj_8, deg_adj_9, deg_adj_10,
           gc1_W, gc1_b, gc2_W, gc2_b,
           bn1_gamma, bn1_beta, bn1_mean, bn1_var,
           dense_W, dense_b,
           bn3_gamma, bn3_beta, bn3_mean, bn3_var):
    del degree_slice
    adjs = [deg_adj_1, deg_adj_2, deg_adj_3, deg_adj_4, deg_adj_5,
            deg_adj_6, deg_adj_7, deg_adj_8, deg_adj_9, deg_adj_10]
    cols = jnp.concatenate([a.T.reshape(-1) for a in adjs])
    cols_r = jnp.concatenate([a.reshape(-1) for a in adjs])

    w1r, w1s = gc1_W[0:2 * MAX_DEG:2], gc1_W[1:2 * MAX_DEG:2]
    b1 = (gc1_b[0:2 * MAX_DEG:2] + gc1_b[1:2 * MAX_DEG:2]).reshape(MAX_DEG, 1, D_HID)
    w2r, w2s = gc2_W[0:2 * MAX_DEG:2], gc2_W[1:2 * MAX_DEG:2]
    b2 = (gc2_b[0:2 * MAX_DEG:2] + gc2_b[1:2 * MAX_DEG:2]).reshape(MAX_DEG, 1, D_HID)
    sc1, sh1 = _bn_fold(bn1_gamma, bn1_beta, bn1_mean, bn1_var)
    sc3, sh3 = _bn_fold(bn3_gamma, bn3_beta, bn3_mean, bn3_var)

    rel1 = _nbr_sum(atom_features, cols)
    y1 = _gc_matmul(atom_features, rel1, w1r, w1s, b1, sc1, sh1)
    p1 = _pool_max(y1, cols_r)
    rel2 = _nbr_sum(p1, cols)
    y2 = _gc_matmul(p1, rel2, w2r, w2s, b2, sc1, sh1)
    p2 = _pool_max(y2, cols_r)
    z = _dense(p2, dense_W, dense_b.reshape(1, -1), sc3, sh3)
    ps, pm = _seg_gather(z, membership)
    out_s, out_m = _assemble(ps, pm)
    out_s = out_s.reshape(8, BATCH, 32).transpose(1, 0, 2).reshape(BATCH, D_DENSE)
    out_m = out_m.reshape(8, 2, BATCH, 16).transpose(2, 0, 1, 3).reshape(BATCH, D_DENSE)
    return jnp.concatenate([out_s, out_m], axis=1)
